# baseline (speedup 1.0000x reference)
import jax
import jax.numpy as jnp
from jax import lax
from jax.experimental import pallas as pl
from jax.experimental.pallas import tpu as pltpu
from jax.experimental.pallas import tpu_sc as plsc

N = 10000
E = 320000
S = 6
L = 2
D = 128
H = 128
O = 128

_NC, _NS = 2, 16
_NW = _NC * _NS
_CH = 128
_KCH = 80
_EPAD = _NW * _KCH * _CH
_NACC = 10112
_ZROWS = _NACC // _NS


_NB = 2
_NHALF = 2
_HKCH = _KCH // _NHALF


def _seg_body(feat_hbm, src_hbm, dst_hbm, zeros_hbm, out_hbm,
              sidx, didx, buf, acc, gs0, gs1, ss0, ss1):
    gsems = (gs0, gs1)
    ssems = (ss0, ss1)
    c = lax.axis_index("c")
    s = lax.axis_index("s")
    wid = s * _NC + c
    pltpu.sync_copy(zeros_hbm.at[pl.ds(0, _ZROWS)],
                    acc.at[pl.ds(s * _ZROWS, _ZROWS)])
    plsc.subcore_barrier()

    def start_gather(g, b):
        pltpu.async_copy(feat_hbm.at[sidx.at[g]], buf.at[b], gsems[b])

    def wait_gather(g, b):
        pltpu.make_async_copy(feat_hbm.at[sidx.at[g]], buf.at[b],
                              gsems[b]).wait()

    for half in range(_NHALF):
        cb = wid * _KCH + half * _HKCH
        pltpu.sync_copy(src_hbm.at[pl.ds(cb, _HKCH)], sidx)
        pltpu.sync_copy(dst_hbm.at[pl.ds(cb, _HKCH)], didx)

        for b in range(_NB):
            start_gather(b, b)

        def gg_body(gg, carry):
            g0 = gg * _NB
            descs = []
            for b in range(_NB):
                wait_gather(g0 + b, b)
                descs.append(pltpu.async_copy(
                    buf.at[b], acc.at[didx.at[g0 + b]], ssems[b], add=True))
            for b in range(_NB):
                descs[b].wait()
                start_gather(g0 + b + _NB, b)
            return carry

        lax.fori_loop(0, _HKCH // _NB - 1, gg_body, 0)

        g0 = _HKCH - _NB
        descs = []
        for b in range(_NB):
            wait_gather(g0 + b, b)
            descs.append(pltpu.async_copy(
                buf.at[b], acc.at[didx.at[g0 + b]], ssems[b], add=True))
        for b in range(_NB):
            descs[b].wait()

    plsc.subcore_barrier()
    pltpu.sync_copy(acc.at[pl.ds(s * _ZROWS, _ZROWS)],
                    out_hbm.at[c, pl.ds(s * _ZROWS, _ZROWS)])


_seg_call = pl.kernel(
    _seg_body,
    out_type=jax.ShapeDtypeStruct((_NC, _NACC, H), jnp.float32),
    mesh=plsc.VectorSubcoreMesh(core_axis_name="c", subcore_axis_name="s"),
    scratch_types=[
        pltpu.VMEM((_HKCH, _CH), jnp.int32),
        pltpu.VMEM((_HKCH, _CH), jnp.int32),
        pltpu.VMEM((_NB, _CH, H), jnp.float32),
        pltpu.VMEM_SHARED((_NACC, H), jnp.float32),
    ] + [pltpu.SemaphoreType.DMA] * (2 * _NB),
)

_BLK = 1000


def _proj_body(h_ref, w_ref, b_ref, o_ref):
    o_ref[...] = jnp.dot(h_ref[...], w_ref[...],
                         preferred_element_type=jnp.float32) + b_ref[...]


def _proj(h, w, b):
    return pl.pallas_call(
        _proj_body,
        grid=(N // _BLK,),
        in_specs=[
            pl.BlockSpec((_BLK, H), lambda i: (i, 0)),
            pl.BlockSpec((H, O), lambda i: (0, 0)),
            pl.BlockSpec((1, O), lambda i: (0, 0)),
        ],
        out_specs=pl.BlockSpec((_BLK, O), lambda i: (i, 0)),
        out_shape=jax.ShapeDtypeStruct((N, O), jnp.float32),
    )(h, w, b.reshape(1, O))


def kernel(x, hidden_states, edge_index, Wx_self, Wx_neigh, bx, Wh_self, Wh_neigh, bh, out_W, out_b):
    src, dst = lax.sort_key_val(edge_index[0], edge_index[1])
    pad = _EPAD - E
    src_p = jnp.concatenate([src, jnp.zeros((pad,), jnp.int32)]).reshape(_NW * _KCH, _CH)
    dst_p = jnp.concatenate([dst, jnp.full((pad,), N, jnp.int32)]).reshape(_NW * _KCH, _CH)
    zeros = jnp.zeros((_ZROWS, H), jnp.float32)

    def seg(feat):
        parts = _seg_call(feat, src_p, dst_p, zeros)
        return parts[0, :N] + parts[1, :N]

    def net(feat, agg, Ws, Wn, b):
        if agg is None:
            agg = seg(feat)
        return feat @ Ws + agg @ Wn + b

    def cell(l, xi, h, x_agg):
        h_agg = seg(h)
        if x_agg is None:
            x_agg = seg(xi)
        r = jax.nn.sigmoid(net(xi, x_agg, Wx_self[l, 0], Wx_neigh[l, 0], bx[l, 0]) + net(h, h_agg, Wh_self[l, 0], Wh_neigh[l, 0], bh[l, 0]))
        u = jax.nn.sigmoid(net(xi, x_agg, Wx_self[l, 1], Wx_neigh[l, 1], bx[l, 1]) + net(h, h_agg, Wh_self[l, 1], Wh_neigh[l, 1], bh[l, 1]))
        h_ = r * h
        c = jnp.tanh(net(xi, x_agg, Wx_self[l, 2], Wx_neigh[l, 2], bx[l, 2]) + net(h_, None, Wh_self[l, 2], Wh_neigh[l, 2], bh[l, 2]))
        return u * h + (1.0 - u) * c

    x_aggs = [seg(x[i]) for i in range(S)]

    hs = [hidden_states[j] for j in range(L)]
    outputs = []
    for i in range(S):
        inp = x[i]
        x_agg = x_aggs[i]
        new_hs = []
        for j in range(L):
            inp = cell(j, inp, hs[j], x_agg)
            new_hs.append(inp)
            x_agg = None
        outputs.append(_proj(inp, out_W, out_b))
        hs = new_hs
    return jnp.stack(outputs), jnp.stack(hs)


# speedup vs baseline: 1.8406x; 1.8406x over previous
import jax
import jax.numpy as jnp
from jax import lax
from jax.experimental import pallas as pl
from jax.experimental.pallas import tpu as pltpu
from jax.experimental.pallas import tpu_sc as plsc

N = 10000
E = 320000
S = 6
L = 2
D = 128
H = 128
O = 128

_NC, _NS = 2, 16
_NW = _NC * _NS
_NPASS = 4
_TROWS = 2560
_TPAD = _NPASS * _TROWS
_CH = 64
_TCH = 48
_TCAP = _TCH * _CH
_QCAP = _NW * _TCAP
_IB = 16
_NB = 2
_NACC = 10112
_ZROWS = _NACC // _NS
_SROWS = _TROWS // _NS


def _seg_body(featp_hbm, lidx_hbm, didx_hbm, zeros_hbm, out_hbm,
              sidx, didx, buf, tbl, acc, gs0, gs1, ss0, ss1):
    gsems = (gs0, gs1)
    ssems = (ss0, ss1)
    c = lax.axis_index("c")
    s = lax.axis_index("s")
    wid = s * _NC + c
    pltpu.sync_copy(zeros_hbm.at[pl.ds(0, _ZROWS)],
                    acc.at[pl.ds(s * _ZROWS, _ZROWS)])

    def start_gather(g, b):
        pltpu.async_copy(tbl.at[sidx.at[g]], buf.at[b], gsems[b])

    def wait_gather(g, b):
        pltpu.make_async_copy(tbl.at[sidx.at[g]], buf.at[b],
                              gsems[b]).wait()

    for q in range(_NPASS):
        plsc.subcore_barrier()
        pltpu.sync_copy(featp_hbm.at[pl.ds(q * _TROWS + s * _SROWS, _SROWS)],
                        tbl.at[pl.ds(s * _SROWS, _SROWS)])
        plsc.subcore_barrier()
        rowbase = (q * _NW + wid) * _TCH
        for blk in range(_TCH // _IB):
            pltpu.sync_copy(lidx_hbm.at[pl.ds(rowbase + blk * _IB, _IB)], sidx)
            pltpu.sync_copy(didx_hbm.at[pl.ds(rowbase + blk * _IB, _IB)], didx)

            for b in range(_NB):
                start_gather(b, b)

            def gg_body(gg, carry):
                g0 = gg * _NB
                descs = []
                for b in range(_NB):
                    wait_gather(g0 + b, b)
                    descs.append(pltpu.async_copy(
                        buf.at[b], acc.at[didx.at[g0 + b]], ssems[b],
                        add=True))
                for b in range(_NB):
                    descs[b].wait()
                    start_gather(g0 + b + _NB, b)
                return carry

            lax.fori_loop(0, _IB // _NB - 1, gg_body, 0)

            g0 = _IB - _NB
            descs = []
            for b in range(_NB):
                wait_gather(g0 + b, b)
                descs.append(pltpu.async_copy(
                    buf.at[b], acc.at[didx.at[g0 + b]], ssems[b], add=True))
            for b in range(_NB):
                descs[b].wait()

    plsc.subcore_barrier()
    pltpu.sync_copy(acc.at[pl.ds(s * _ZROWS, _ZROWS)],
                    out_hbm.at[c, pl.ds(s * _ZROWS, _ZROWS)])


_seg_call = pl.kernel(
    _seg_body,
    out_type=jax.ShapeDtypeStruct((_NC, _NACC, H), jnp.float32),
    mesh=plsc.VectorSubcoreMesh(core_axis_name="c", subcore_axis_name="s"),
    scratch_types=[
        pltpu.VMEM((_IB, _CH), jnp.int32),
        pltpu.VMEM((_IB, _CH), jnp.int32),
        pltpu.VMEM((_NB, _CH, H), jnp.float32),
        pltpu.VMEM_SHARED((_TROWS, H), jnp.float32),
        pltpu.VMEM_SHARED((_NACC, H), jnp.float32),
    ] + [pltpu.SemaphoreType.DMA] * (2 * _NB),
)

_BLK = 1000


def _proj_body(h_ref, w_ref, b_ref, o_ref):
    o_ref[...] = jnp.dot(h_ref[...], w_ref[...],
                         preferred_element_type=jnp.float32) + b_ref[...]


def _proj(h, w, b):
    return pl.pallas_call(
        _proj_body,
        grid=(N // _BLK,),
        in_specs=[
            pl.BlockSpec((_BLK, H), lambda i: (i, 0)),
            pl.BlockSpec((H, O), lambda i: (0, 0)),
            pl.BlockSpec((1, O), lambda i: (0, 0)),
        ],
        out_specs=pl.BlockSpec((_BLK, O), lambda i: (i, 0)),
        out_shape=jax.ShapeDtypeStruct((N, O), jnp.float32),
    )(h, w, b.reshape(1, O))


def _route_edges(edge_index):
    src_s, dst_s = lax.sort_key_val(edge_index[0], edge_index[1])
    qb = jnp.array([0, _TROWS, 2 * _TROWS, 3 * _TROWS], jnp.int32)
    qstart = jnp.searchsorted(src_s, qb).astype(jnp.int32)
    qend = jnp.concatenate([qstart[1:], jnp.array([E], jnp.int32)])
    qcnt = qend - qstart
    srcp = jnp.concatenate([src_s, jnp.zeros((_QCAP,), jnp.int32)])
    dstp = jnp.concatenate([dst_s, jnp.full((_QCAP,), N, jnp.int32)])
    pos = jnp.arange(_QCAP, dtype=jnp.int32)
    lis, dis = [], []
    for q in range(_NPASS):
        sl = lax.dynamic_slice(srcp, (qstart[q],), (_QCAP,))
        dl = lax.dynamic_slice(dstp, (qstart[q],), (_QCAP,))
        valid = pos < qcnt[q]
        li = jnp.where(valid, sl - q * _TROWS, 0)
        di = jnp.where(valid, dl, N)
        lis.append(li.reshape(_TCAP, _NW).T.reshape(_NW, _TCH, _CH))
        dis.append(di.reshape(_TCAP, _NW).T.reshape(_NW, _TCH, _CH))
    lidx = jnp.stack(lis).reshape(_NPASS * _NW * _TCH, _CH)
    didx = jnp.stack(dis).reshape(_NPASS * _NW * _TCH, _CH)
    return lidx, didx


def kernel(x, hidden_states, edge_index, Wx_self, Wx_neigh, bx, Wh_self, Wh_neigh, bh, out_W, out_b):
    lidx, didx = _route_edges(edge_index)
    zeros = jnp.zeros((_ZROWS, H), jnp.float32)
    tpad = jnp.zeros((_TPAD - N, H), jnp.float32)

    def seg(feat):
        featp = jnp.concatenate([feat, tpad])
        parts = _seg_call(featp, lidx, didx, zeros)
        return parts[0, :N] + parts[1, :N]

    def net(feat, agg, Ws, Wn, b):
        if agg is None:
            agg = seg(feat)
        return feat @ Ws + agg @ Wn + b

    def cell(l, xi, h, x_agg):
        h_agg = seg(h)
        if x_agg is None:
            x_agg = seg(xi)
        r = jax.nn.sigmoid(net(xi, x_agg, Wx_self[l, 0], Wx_neigh[l, 0], bx[l, 0]) + net(h, h_agg, Wh_self[l, 0], Wh_neigh[l, 0], bh[l, 0]))
        u = jax.nn.sigmoid(net(xi, x_agg, Wx_self[l, 1], Wx_neigh[l, 1], bx[l, 1]) + net(h, h_agg, Wh_self[l, 1], Wh_neigh[l, 1], bh[l, 1]))
        h_ = r * h
        c = jnp.tanh(net(xi, x_agg, Wx_self[l, 2], Wx_neigh[l, 2], bx[l, 2]) + net(h_, None, Wh_self[l, 2], Wh_neigh[l, 2], bh[l, 2]))
        return u * h + (1.0 - u) * c

    x_aggs = [seg(x[i]) for i in range(S)]

    hs = [hidden_states[j] for j in range(L)]
    outputs = []
    for i in range(S):
        inp = x[i]
        x_agg = x_aggs[i]
        new_hs = []
        for j in range(L):
            inp = cell(j, inp, hs[j], x_agg)
            new_hs.append(inp)
            x_agg = None
        outputs.append(_proj(inp, out_W, out_b))
        hs = new_hs
    return jnp.stack(outputs), jnp.stack(hs)


# speedup vs baseline: 2.2010x; 1.1958x over previous
import jax
import jax.numpy as jnp
from jax import lax
from jax.experimental import pallas as pl
from jax.experimental.pallas import tpu as pltpu
from jax.experimental.pallas import tpu_sc as plsc

N = 10000
E = 320000
S = 6
L = 2
D = 128
H = 128
O = 128

_NC, _NS = 2, 16
_NPASS = 4
_TROWS = 2560
_TPAD = _NPASS * _TROWS
_HALF0 = 5056
_NACC = 5120
_ZROWS = _NACC // _NS
_DUMP = 5056
_CH = 128
_TCH = 24
_TCAP = _TCH * _CH
_GCAP = _NS * _TCAP
_NB = 3
_SROWS = _TROWS // _NS


def _seg_body(featp_hbm, lidx_hbm, didx_hbm, zeros_hbm, out_hbm,
              sidx, didx, buf, tbl, acc, gs0, gs1, gs2, ss0, ss1, ss2):
    gsems = (gs0, gs1, gs2)
    ssems = (ss0, ss1, ss2)
    c = lax.axis_index("c")
    s = lax.axis_index("s")
    pltpu.sync_copy(zeros_hbm.at[pl.ds(0, _ZROWS)],
                    acc.at[pl.ds(s * _ZROWS, _ZROWS)])

    def start_gather(g, b):
        pltpu.async_copy(tbl.at[sidx.at[g]], buf.at[b], gsems[b])

    def wait_gather(g, b):
        pltpu.make_async_copy(tbl.at[sidx.at[g]], buf.at[b],
                              gsems[b]).wait()

    for q in range(_NPASS):
        plsc.subcore_barrier()
        pltpu.sync_copy(featp_hbm.at[pl.ds(q * _TROWS + s * _SROWS, _SROWS)],
                        tbl.at[pl.ds(s * _SROWS, _SROWS)])
        plsc.subcore_barrier()
        rowbase = ((c * _NPASS + q) * _NS + s) * _TCH
        pltpu.sync_copy(lidx_hbm.at[pl.ds(rowbase, _TCH)], sidx)
        pltpu.sync_copy(didx_hbm.at[pl.ds(rowbase, _TCH)], didx)

        for b in range(_NB):
            start_gather(b, b)

        def gg_body(gg, carry):
            g0 = gg * _NB
            descs = []
            for b in range(_NB):
                wait_gather(g0 + b, b)
                descs.append(pltpu.async_copy(
                    buf.at[b], acc.at[didx.at[g0 + b]], ssems[b], add=True))
            for b in range(_NB):
                descs[b].wait()
                start_gather(g0 + b + _NB, b)
            return carry

        lax.fori_loop(0, _TCH // _NB - 1, gg_body, 0)

        g0 = _TCH - _NB
        descs = []
        for b in range(_NB):
            wait_gather(g0 + b, b)
            descs.append(pltpu.async_copy(
                buf.at[b], acc.at[didx.at[g0 + b]], ssems[b], add=True))
        for b in range(_NB):
            descs[b].wait()

    plsc.subcore_barrier()
    pltpu.sync_copy(acc.at[pl.ds(s * _ZROWS, _ZROWS)],
                    out_hbm.at[c, pl.ds(s * _ZROWS, _ZROWS)])


_seg_call = pl.kernel(
    _seg_body,
    out_type=jax.ShapeDtypeStruct((_NC, _NACC, H), jnp.float32),
    mesh=plsc.VectorSubcoreMesh(core_axis_name="c", subcore_axis_name="s"),
    scratch_types=[
        pltpu.VMEM((_TCH, _CH), jnp.int32),
        pltpu.VMEM((_TCH, _CH), jnp.int32),
        pltpu.VMEM((_NB, _CH, H), jnp.float32),
        pltpu.VMEM_SHARED((_TROWS, H), jnp.float32),
        pltpu.VMEM_SHARED((_NACC, H), jnp.float32),
    ] + [pltpu.SemaphoreType.DMA] * (2 * _NB),
)

_BLK = 1000


def _proj_body(h_ref, w_ref, b_ref, o_ref):
    o_ref[...] = jnp.dot(h_ref[...], w_ref[...],
                         preferred_element_type=jnp.float32) + b_ref[...]


def _proj(h, w, b):
    return pl.pallas_call(
        _proj_body,
        grid=(N // _BLK,),
        in_specs=[
            pl.BlockSpec((_BLK, H), lambda i: (i, 0)),
            pl.BlockSpec((H, O), lambda i: (0, 0)),
            pl.BlockSpec((1, O), lambda i: (0, 0)),
        ],
        out_specs=pl.BlockSpec((_BLK, O), lambda i: (i, 0)),
        out_shape=jax.ShapeDtypeStruct((N, O), jnp.float32),
    )(h, w, b.reshape(1, O))


def _route_edges(edge_index):
    src0 = edge_index[0]
    dst0 = edge_index[1]
    key = (dst0 >= _HALF0).astype(jnp.int32) * _NPASS + src0 // _TROWS
    key_s, src_s, dst_s = lax.sort((key, src0, dst0), num_keys=1)
    starts = jnp.searchsorted(key_s, jnp.arange(9, dtype=jnp.int32)).astype(jnp.int32)
    cnt = starts[1:] - starts[:-1]
    srcp = jnp.concatenate([src_s, jnp.zeros((_GCAP,), jnp.int32)])
    dstp = jnp.concatenate([dst_s, jnp.zeros((_GCAP,), jnp.int32)])
    pos = jnp.arange(_GCAP, dtype=jnp.int32)
    lis, dis = [], []
    for g in range(_NC * _NPASS):
        c, q = g // _NPASS, g % _NPASS
        sl = lax.dynamic_slice(srcp, (starts[g],), (_GCAP,))
        dl = lax.dynamic_slice(dstp, (starts[g],), (_GCAP,))
        valid = pos < cnt[g]
        li = jnp.where(valid, sl - q * _TROWS, 0)
        di = jnp.where(valid, dl - c * _HALF0, _DUMP)
        lis.append(li.reshape(_TCAP, _NS).T.reshape(_NS, _TCH, _CH))
        dis.append(di.reshape(_TCAP, _NS).T.reshape(_NS, _TCH, _CH))
    lidx = jnp.stack(lis).reshape(_NC * _NPASS * _NS * _TCH, _CH)
    didx = jnp.stack(dis).reshape(_NC * _NPASS * _NS * _TCH, _CH)
    return lidx, didx


def kernel(x, hidden_states, edge_index, Wx_self, Wx_neigh, bx, Wh_self, Wh_neigh, bh, out_W, out_b):
    lidx, didx = _route_edges(edge_index)
    zeros = jnp.zeros((_ZROWS, H), jnp.float32)
    tpad = jnp.zeros((_TPAD - N, H), jnp.float32)

    def seg(feat):
        featp = jnp.concatenate([feat, tpad])
        parts = _seg_call(featp, lidx, didx, zeros)
        return jnp.concatenate([parts[0, :_HALF0], parts[1, :N - _HALF0]])

    def net(feat, agg, Ws, Wn, b):
        if agg is None:
            agg = seg(feat)
        return feat @ Ws + agg @ Wn + b

    def cell(l, xi, h, x_agg):
        h_agg = seg(h)
        if x_agg is None:
            x_agg = seg(xi)
        r = jax.nn.sigmoid(net(xi, x_agg, Wx_self[l, 0], Wx_neigh[l, 0], bx[l, 0]) + net(h, h_agg, Wh_self[l, 0], Wh_neigh[l, 0], bh[l, 0]))
        u = jax.nn.sigmoid(net(xi, x_agg, Wx_self[l, 1], Wx_neigh[l, 1], bx[l, 1]) + net(h, h_agg, Wh_self[l, 1], Wh_neigh[l, 1], bh[l, 1]))
        h_ = r * h
        c = jnp.tanh(net(xi, x_agg, Wx_self[l, 2], Wx_neigh[l, 2], bx[l, 2]) + net(h_, None, Wh_self[l, 2], Wh_neigh[l, 2], bh[l, 2]))
        return u * h + (1.0 - u) * c

    x_aggs = [seg(x[i]) for i in range(S)]

    hs = [hidden_states[j] for j in range(L)]
    outputs = []
    for i in range(S):
        inp = x[i]
        x_agg = x_aggs[i]
        new_hs = []
        for j in range(L):
            inp = cell(j, inp, hs[j], x_agg)
            new_hs.append(inp)
            x_agg = None
        outputs.append(_proj(inp, out_W, out_b))
        hs = new_hs
    return jnp.stack(outputs), jnp.stack(hs)


# speedup vs baseline: 2.6173x; 1.1891x over previous
import jax
import jax.numpy as jnp
from jax import lax
from jax.experimental import pallas as pl
from jax.experimental.pallas import tpu as pltpu
from jax.experimental.pallas import tpu_sc as plsc

N = 10000
E = 320000
S = 6
L = 2
D = 128
H = 128
O = 128

_NC, _NS = 2, 16
_NPASS = 4
_TROWS = 2560
_TPAD = _NPASS * _TROWS
_HALF0 = 5056
_NACC = 5120
_ZROWS = _NACC // _NS
_DUMP = 5056
_CH = 128
_TCH = 24
_TCAP = _TCH * _CH
_GCAP = _NS * _TCAP
_NB = 3
_SROWS = _TROWS // _NS


def _seg_body(featp_hbm, lidx_hbm, didx_hbm, zeros_hbm, tgrp_hbm, out_hbm,
              sidx, didx, tv, buf, tbl, acc, gs0, gs1, gs2, ss0, ss1, ss2):
    gsems = (gs0, gs1, gs2)
    ssems = (ss0, ss1, ss2)
    c = lax.axis_index("c")
    s = lax.axis_index("s")
    pltpu.sync_copy(tgrp_hbm, tv)
    pltpu.sync_copy(zeros_hbm.at[pl.ds(0, _ZROWS)],
                    acc.at[pl.ds(s * _ZROWS, _ZROWS)])

    def start_gather(g, b):
        pltpu.async_copy(tbl.at[sidx.at[g]], buf.at[b], gsems[b])

    def wait_gather(g, b):
        pltpu.make_async_copy(tbl.at[sidx.at[g]], buf.at[b],
                              gsems[b]).wait()

    for q in range(_NPASS):
        plsc.subcore_barrier()
        pltpu.sync_copy(featp_hbm.at[pl.ds(q * _TROWS + s * _SROWS, _SROWS)],
                        tbl.at[pl.ds(s * _SROWS, _SROWS)])
        plsc.subcore_barrier()
        rowbase = ((c * _NPASS + q) * _NS + s) * _TCH
        pltpu.sync_copy(lidx_hbm.at[pl.ds(rowbase, _TCH)], sidx)
        pltpu.sync_copy(didx_hbm.at[pl.ds(rowbase, _TCH)], didx)
        tvv = tv[...]
        tcur = jnp.where(c == 0, tvv[q], tvv[_NPASS + q])

        for b in range(_NB):
            start_gather(b, b)

        def gg_body(gg, carry):
            g0 = gg * _NB
            descs = []
            for b in range(_NB):
                wait_gather(g0 + b, b)
                descs.append(pltpu.async_copy(
                    buf.at[b], acc.at[didx.at[g0 + b]], ssems[b], add=True))
            for b in range(_NB):
                descs[b].wait()
                start_gather(g0 + b + _NB, b)
            return carry

        lax.fori_loop(0, tcur - 1, gg_body, 0)

        g0 = (tcur - 1) * _NB
        descs = []
        for b in range(_NB):
            wait_gather(g0 + b, b)
            descs.append(pltpu.async_copy(
                buf.at[b], acc.at[didx.at[g0 + b]], ssems[b], add=True))
        for b in range(_NB):
            descs[b].wait()

    plsc.subcore_barrier()
    pltpu.sync_copy(acc.at[pl.ds(s * _ZROWS, _ZROWS)],
                    out_hbm.at[c, pl.ds(s * _ZROWS, _ZROWS)])


_seg_call = pl.kernel(
    _seg_body,
    out_type=jax.ShapeDtypeStruct((_NC, _NACC, H), jnp.float32),
    mesh=plsc.VectorSubcoreMesh(core_axis_name="c", subcore_axis_name="s"),
    scratch_types=[
        pltpu.VMEM((_TCH, _CH), jnp.int32),
        pltpu.VMEM((_TCH, _CH), jnp.int32),
        pltpu.VMEM((16,), jnp.int32),
        pltpu.VMEM((_NB, _CH, H), jnp.float32),
        pltpu.VMEM_SHARED((_TROWS, H), jnp.float32),
        pltpu.VMEM_SHARED((_NACC, H), jnp.float32),
    ] + [pltpu.SemaphoreType.DMA] * (2 * _NB),
)

_BLK = 1000


def _proj_body(h_ref, w_ref, b_ref, o_ref):
    o_ref[...] = jnp.dot(h_ref[...], w_ref[...],
                         preferred_element_type=jnp.float32) + b_ref[...]


def _proj(h, w, b):
    return pl.pallas_call(
        _proj_body,
        grid=(N // _BLK,),
        in_specs=[
            pl.BlockSpec((_BLK, H), lambda i: (i, 0)),
            pl.BlockSpec((H, O), lambda i: (0, 0)),
            pl.BlockSpec((1, O), lambda i: (0, 0)),
        ],
        out_specs=pl.BlockSpec((_BLK, O), lambda i: (i, 0)),
        out_shape=jax.ShapeDtypeStruct((N, O), jnp.float32),
    )(h, w, b.reshape(1, O))


def _route_edges(edge_index):
    src0 = edge_index[0]
    dst0 = edge_index[1]
    key = (dst0 >= _HALF0).astype(jnp.int32) * _NPASS + src0 // _TROWS
    key_s, src_s, dst_s = lax.sort((key, src0, dst0), num_keys=1)
    starts = jnp.searchsorted(key_s, jnp.arange(9, dtype=jnp.int32)).astype(jnp.int32)
    cnt = starts[1:] - starts[:-1]
    srcp = jnp.concatenate([src_s, jnp.zeros((_GCAP,), jnp.int32)])
    dstp = jnp.concatenate([dst_s, jnp.zeros((_GCAP,), jnp.int32)])
    pos = jnp.arange(_GCAP, dtype=jnp.int32)
    lis, dis = [], []
    for g in range(_NC * _NPASS):
        c, q = g // _NPASS, g % _NPASS
        sl = lax.dynamic_slice(srcp, (starts[g],), (_GCAP,))
        dl = lax.dynamic_slice(dstp, (starts[g],), (_GCAP,))
        valid = pos < cnt[g]
        li = jnp.where(valid, sl - q * _TROWS, 0)
        di = jnp.where(valid, dl - c * _HALF0, _DUMP)
        lis.append(li.reshape(_TCAP, _NS).T.reshape(_NS, _TCH, _CH))
        dis.append(di.reshape(_TCAP, _NS).T.reshape(_NS, _TCH, _CH))
    lidx = jnp.stack(lis).reshape(_NC * _NPASS * _NS * _TCH, _CH)
    didx = jnp.stack(dis).reshape(_NC * _NPASS * _NS * _TCH, _CH)
    tcnt = (cnt + _NS - 1) // _NS
    nch = (tcnt + _CH - 1) // _CH
    tgrp = jnp.maximum(1, (nch + _NB - 1) // _NB).astype(jnp.int32)
    tgrp = jnp.concatenate([tgrp, jnp.ones((16 - _NC * _NPASS,), jnp.int32)])
    return lidx, didx, tgrp


def kernel(x, hidden_states, edge_index, Wx_self, Wx_neigh, bx, Wh_self, Wh_neigh, bh, out_W, out_b):
    lidx, didx, tgrp = _route_edges(edge_index)
    zeros = jnp.zeros((_ZROWS, H), jnp.float32)
    tpad = jnp.zeros((_TPAD - N, H), jnp.float32)

    def seg(feat):
        featp = jnp.concatenate([feat, tpad])
        parts = _seg_call(featp, lidx, didx, zeros, tgrp)
        return jnp.concatenate([parts[0, :_HALF0], parts[1, :N - _HALF0]])

    def net(feat, agg, Ws, Wn, b):
        if agg is None:
            agg = seg(feat)
        return feat @ Ws + agg @ Wn + b

    def cell(l, xi, h, x_agg):
        h_agg = seg(h)
        if x_agg is None:
            x_agg = seg(xi)
        r = jax.nn.sigmoid(net(xi, x_agg, Wx_self[l, 0], Wx_neigh[l, 0], bx[l, 0]) + net(h, h_agg, Wh_self[l, 0], Wh_neigh[l, 0], bh[l, 0]))
        u = jax.nn.sigmoid(net(xi, x_agg, Wx_self[l, 1], Wx_neigh[l, 1], bx[l, 1]) + net(h, h_agg, Wh_self[l, 1], Wh_neigh[l, 1], bh[l, 1]))
        h_ = r * h
        c = jnp.tanh(net(xi, x_agg, Wx_self[l, 2], Wx_neigh[l, 2], bx[l, 2]) + net(h_, None, Wh_self[l, 2], Wh_neigh[l, 2], bh[l, 2]))
        return u * h + (1.0 - u) * c

    x_aggs = [seg(x[i]) for i in range(S)]

    hs = [hidden_states[j] for j in range(L)]
    outputs = []
    for i in range(S):
        inp = x[i]
        x_agg = x_aggs[i]
        new_hs = []
        for j in range(L):
            inp = cell(j, inp, hs[j], x_agg)
            new_hs.append(inp)
            x_agg = None
        outputs.append(_proj(inp, out_W, out_b))
        hs = new_hs
    return jnp.stack(outputs), jnp.stack(hs)
